# baseline (speedup 1.0000x reference)
import functools
import jax
import jax.numpy as jnp
from jax import lax
from jax.experimental import pallas as pl
from jax.experimental.pallas import tpu as pltpu
from jax.experimental.pallas import tpu_sc as plsc

_R = 256
_B, _S, _L = 4, 8192, 2048
_NW = 32
_WL = 128


def _sc_body(x_hbm, out_hbm, ibuf, obuf):
    wid = lax.axis_index("s") * 2 + lax.axis_index("c")
    lane0 = (wid % 16) * _WL
    b0 = (wid // 16) * 2

    for bi in range(2):
        b = b0 + bi

        def chunk(ci, accs):
            r0 = ci * _R
            pltpu.sync_copy(x_hbm.at[b, pl.ds(r0, _R), pl.ds(lane0, _WL)], ibuf)

            def row(r, accs):
                new = []
                for c in range(_WL // 16):
                    a = accs[c] + ibuf[r, pl.ds(c * 16, 16)]
                    obuf[r, pl.ds(c * 16, 16)] = a
                    new.append(a)
                return tuple(new)

            accs = lax.fori_loop(0, _R, row, accs)
            pltpu.sync_copy(obuf, out_hbm.at[b, pl.ds(r0, _R), pl.ds(lane0, _WL)])
            return accs

        zeros = tuple(jnp.zeros((16,), jnp.float32) for _ in range(_WL // 16))
        lax.fori_loop(0, _S // _R, chunk, zeros)


def kernel(x):
    k = pl.kernel(
        _sc_body,
        out_type=jax.ShapeDtypeStruct((_B, _S, _L), jnp.float32),
        mesh=plsc.VectorSubcoreMesh(core_axis_name="c", subcore_axis_name="s"),
        scratch_types=[
            pltpu.VMEM((_R, _WL), jnp.float32),
            pltpu.VMEM((_R, _WL), jnp.float32),
        ],
    )
    return k(x)


# speedup vs baseline: 1.5552x; 1.5552x over previous
import jax
import jax.numpy as jnp
from jax import lax
from jax.experimental import pallas as pl
from jax.experimental.pallas import tpu as pltpu
from jax.experimental.pallas import tpu_sc as plsc

_R = 128
_B, _S, _L = 4, 8192, 2048
_WL = 128
_NC = _S // _R
_NV = _WL // 16


def _sc_body(x_hbm, out_hbm, ib0, ib1, ob0, ob1, si0, si1, so0, so1):
    wid = lax.axis_index("s") * 2 + lax.axis_index("c")
    lane0 = (wid % 16) * _WL
    b0 = (wid // 16) * 2

    def in_cp(b, ci, buf, sem):
        return pltpu.make_async_copy(
            x_hbm.at[b, pl.ds(ci * _R, _R), pl.ds(lane0, _WL)], buf, sem
        )

    def out_cp(b, ci, buf, sem):
        return pltpu.make_async_copy(
            buf, out_hbm.at[b, pl.ds(ci * _R, _R), pl.ds(lane0, _WL)], sem
        )

    def scan_chunk(ibuf, obuf, accs):
        def row(r, accs):
            new = []
            for c in range(_NV):
                a = accs[c] + ibuf[r, pl.ds(c * 16, 16)]
                obuf[r, pl.ds(c * 16, 16)] = a
                new.append(a)
            return tuple(new)

        return lax.fori_loop(0, _R, row, accs)

    for bi in range(2):
        b = b0 + bi
        in_cp(b, 0, ib0, si0).start()

        def pair(i, accs):
            c0 = 2 * i
            in_cp(b, c0 + 1, ib1, si1).start()
            in_cp(b, c0, ib0, si0).wait()

            @pl.when(i > 0)
            def _():
                out_cp(b, c0, ob0, so0).wait()

            accs = scan_chunk(ib0, ob0, accs)
            out_cp(b, c0, ob0, so0).start()
            nxt = jnp.minimum(c0 + 2, _NC - 1)
            in_cp(b, nxt, ib0, si0).start()
            in_cp(b, c0 + 1, ib1, si1).wait()

            @pl.when(i > 0)
            def _():
                out_cp(b, c0 + 1, ob1, so1).wait()

            accs = scan_chunk(ib1, ob1, accs)
            out_cp(b, c0 + 1, ob1, so1).start()
            return accs

        zeros = tuple(jnp.zeros((16,), jnp.float32) for _ in range(_NV))
        lax.fori_loop(0, _NC // 2, pair, zeros)
        in_cp(b, _NC - 1, ib0, si0).wait()
        out_cp(b, _NC - 2, ob0, so0).wait()
        out_cp(b, _NC - 1, ob1, so1).wait()


def kernel(x):
    k = pl.kernel(
        _sc_body,
        out_type=jax.ShapeDtypeStruct((_B, _S, _L), jnp.float32),
        mesh=plsc.VectorSubcoreMesh(core_axis_name="c", subcore_axis_name="s"),
        scratch_types=[
            pltpu.VMEM((_R, _WL), jnp.float32),
            pltpu.VMEM((_R, _WL), jnp.float32),
            pltpu.VMEM((_R, _WL), jnp.float32),
            pltpu.VMEM((_R, _WL), jnp.float32),
            pltpu.SemaphoreType.DMA,
            pltpu.SemaphoreType.DMA,
            pltpu.SemaphoreType.DMA,
            pltpu.SemaphoreType.DMA,
        ],
    )
    return k(x)
